# Initial kernel scaffold; baseline (speedup 1.0000x reference)
import jax
import jax.numpy as jnp
from jax.experimental import pallas as pl


def kernel(x, adj_indices, adj_values, W):
    raise NotImplementedError("write your pallas kernel here")


# speedup vs baseline: 4.4340x; 4.4340x over previous
import functools

import jax
import jax.numpy as jnp
from jax import lax
from jax.experimental import pallas as pl
from jax.experimental.pallas import tpu as pltpu
from jax.experimental.pallas import tpu_sc as plsc

N = 10000
N_PAD = 10240
E = 320000
D_IN = 128
D_OUT = 128

NC = 2
NS = 16
LANES = 16

EDGES_PER_TILE = E // (NC * NS)
CHUNK = 80
NCHUNKS = EDGES_PER_TILE // CHUNK
ROWS_PER_TILE = N_PAD // NS
ZBLK = 128

MM_BLK = 1024


def _matmul_body(x_ref, w_ref, o_ref):
    o_ref[...] = jnp.dot(x_ref[...], w_ref[...], preferred_element_type=jnp.float32)


def _tc_matmul(x, W):
    return pl.pallas_call(
        _matmul_body,
        grid=(N_PAD // MM_BLK,),
        in_specs=[
            pl.BlockSpec((MM_BLK, D_IN), lambda i: (i, 0)),
            pl.BlockSpec((D_IN, D_OUT), lambda i: (0, 0)),
        ],
        out_specs=pl.BlockSpec((MM_BLK, D_OUT), lambda i: (i, 0)),
        out_shape=jax.ShapeDtypeStruct((N_PAD, D_OUT), jnp.float32),
    )(x, W)


def _add_body(a_ref, b_ref, o_ref):
    o_ref[...] = a_ref[...] + b_ref[...]


def _tc_add(a, b):
    return pl.pallas_call(
        _add_body,
        grid=(N_PAD // MM_BLK,),
        in_specs=[
            pl.BlockSpec((MM_BLK, D_OUT), lambda i: (i, 0)),
            pl.BlockSpec((MM_BLK, D_OUT), lambda i: (i, 0)),
        ],
        out_specs=pl.BlockSpec((MM_BLK, D_OUT), lambda i: (i, 0)),
        out_shape=jax.ShapeDtypeStruct((N_PAD, D_OUT), jnp.float32),
    )(a, b)


def _bcast_lane(v, i):
    idx = jnp.full((LANES,), i, dtype=jnp.int32)
    return lax.gather(
        v,
        idx[:, None],
        dimension_numbers=lax.GatherDimensionNumbers(
            offset_dims=(), collapsed_slice_dims=(0,), start_index_map=(0,)
        ),
        slice_sizes=(1,),
        mode=lax.GatherScatterMode.PROMISE_IN_BOUNDS,
    )


def _sc_spmm_body(
    ps, rows_hbm, cols_hbm, vals_hbm, out0, out1,
    cols_v, rows_v, vals_v, buf, zbuf, acc, sem
):
    c = lax.axis_index("c")
    s = lax.axis_index("s")

    zero16 = jnp.zeros((LANES,), jnp.float32)

    def zrow(i, carry):
        for j in range(D_OUT // LANES):
            zbuf[i, pl.ds(j * LANES, LANES)] = zero16
        return carry

    lax.fori_loop(0, ZBLK, zrow, 0)
    row0 = s * ROWS_PER_TILE
    for b in range(ROWS_PER_TILE // ZBLK):
        pltpu.sync_copy(zbuf, acc.at[pl.ds(row0 + b * ZBLK, ZBLK)])
    plsc.subcore_barrier()

    ebase = (c * NS + s) * EDGES_PER_TILE

    def body(it, carry):
        base = ebase + it * CHUNK
        pltpu.sync_copy(cols_hbm.at[pl.ds(base, CHUNK)], cols_v)
        pltpu.sync_copy(rows_hbm.at[pl.ds(base, CHUNK)], rows_v)
        pltpu.sync_copy(vals_hbm.at[pl.ds(base, CHUNK)], vals_v)

        pltpu.async_copy(ps.at[cols_v], buf, sem).wait()

        for g in range(CHUNK // LANES):
            vv = vals_v[pl.ds(g * LANES, LANES)]
            for i in range(LANES):
                e = g * LANES + i
                vb = _bcast_lane(vv, i)
                for j in range(D_OUT // LANES):
                    sl = pl.ds(j * LANES, LANES)
                    buf[e, sl] = buf[e, sl] * vb

        pltpu.sync_copy(buf, acc.at[rows_v], add=True)
        return carry

    lax.fori_loop(0, NCHUNKS, body, 0)
    plsc.subcore_barrier()

    @pl.when(c == 0)
    def _():
        for b in range(ROWS_PER_TILE // ZBLK):
            r = row0 + b * ZBLK
            pltpu.sync_copy(acc.at[pl.ds(r, ZBLK)], out0.at[pl.ds(r, ZBLK)])

    @pl.when(c == 1)
    def _():
        for b in range(ROWS_PER_TILE // ZBLK):
            r = row0 + b * ZBLK
            pltpu.sync_copy(acc.at[pl.ds(r, ZBLK)], out1.at[pl.ds(r, ZBLK)])


_sc_spmm = functools.partial(
    pl.kernel,
    mesh=plsc.VectorSubcoreMesh(core_axis_name="c", subcore_axis_name="s"),
    out_type=[
        jax.ShapeDtypeStruct((N_PAD, D_OUT), jnp.float32),
        jax.ShapeDtypeStruct((N_PAD, D_OUT), jnp.float32),
    ],
    scratch_types=[
        pltpu.VMEM((CHUNK,), jnp.int32),
        pltpu.VMEM((CHUNK,), jnp.int32),
        pltpu.VMEM((CHUNK,), jnp.float32),
        pltpu.VMEM((CHUNK, D_OUT), jnp.float32),
        pltpu.VMEM((ZBLK, D_OUT), jnp.float32),
        pltpu.VMEM_SHARED((N_PAD, D_OUT), jnp.float32),
        pltpu.SemaphoreType.DMA,
    ],
)(_sc_spmm_body)


def kernel(x, adj_indices, adj_values, W):
    x_pad = jnp.pad(x, ((0, N_PAD - N), (0, 0)))
    ps = _tc_matmul(x_pad, W)
    rows = adj_indices[0]
    cols = adj_indices[1]
    p0, p1 = _sc_spmm(ps, rows, cols, adj_values)
    return _tc_add(p0, p1)[:N]
